# baseline (speedup 1.0000x reference)
import dataclasses
import functools

import jax
import jax.numpy as jnp
from jax import lax
from jax.experimental import pallas as pl
from jax.experimental.pallas import tpu as pltpu
from jax.experimental.pallas import tpu_sc as plsc

N = 10000
NS = 32
DIN = 128
DOUT = 128
ALPHA = 0.2

NTILES = 32
NPAD = 10240
NODES_PER_TILE = NPAD // NTILES
PHASES = 2
NODES_PER_PHASE = NODES_PER_TILE // PHASES
CHUNK = 4
ROWS_PER_CHUNK = CHUNK * NS
NCHUNKS = NODES_PER_PHASE // CHUNK

_f32 = jnp.float32
_i32 = jnp.int32


def _tc_body(x_ref, w_ref, ap_ref, h_ref, e12_ref, eluh_ref):
    h = jnp.dot(x_ref[...], w_ref[...], preferred_element_type=_f32)
    e12 = jnp.dot(h, ap_ref[...], preferred_element_type=_f32)
    h_ref[...] = h
    e12_ref[...] = e12
    eluh_ref[...] = jnp.where(h > 0, h, jnp.exp(h) - 1.0)


def _tc_stage(x, w, ap):
    bm = 2000
    grid = (N // bm,)
    return pl.pallas_call(
        _tc_body,
        grid=grid,
        in_specs=[
            pl.BlockSpec((bm, DIN), lambda i: (i, 0)),
            pl.BlockSpec((DIN, DOUT), lambda i: (0, 0)),
            pl.BlockSpec((DOUT, 16), lambda i: (0, 0)),
        ],
        out_specs=[
            pl.BlockSpec((bm, DOUT), lambda i: (i, 0)),
            pl.BlockSpec((bm, 16), lambda i: (i, 0)),
            pl.BlockSpec((bm, DOUT), lambda i: (i, 0)),
        ],
        out_shape=[
            jax.ShapeDtypeStruct((N, DOUT), _f32),
            jax.ShapeDtypeStruct((N, 16), _f32),
            jax.ShapeDtypeStruct((N, DOUT), _f32),
        ],
    )(x, w, ap)


def _splat_i32(v):
    return jnp.full((16,), v, dtype=_i32)


def _sc_compute_chunk(rows, idx_v, e1_v, e2_v, watt, outb, chunk, phase, b):
    for l in range(CHUNK):
        nip = chunk * CHUNK + l
        nb0 = idx_v[pl.ds(nip * NS, 16)]
        nb1 = idx_v[pl.ds(nip * NS + 16, 16)]
        ev0 = plsc.load_gather(e2_v, [nb0])
        ev1 = plsc.load_gather(e2_v, [nb1])
        e1s = plsc.load_gather(e1_v,
                               [_splat_i32(phase * NODES_PER_PHASE + nip)])
        z0 = e1s + ev0
        z1 = e1s + ev1
        z0 = jnp.maximum(z0, ALPHA * z0)
        z1 = jnp.maximum(z1, ALPHA * z1)
        w0 = jnp.exp(z0)
        w1 = jnp.exp(z1)
        den = jnp.sum(w0 + w1)
        watt[pl.ds(0, 16)] = w0
        watt[pl.ds(16, 16)] = w1
        den_v = jnp.full((16,), den, dtype=_f32)

        def body(s, acc):
            ws = plsc.load_gather(watt, [_splat_i32(s)])
            r = l * NS + s
            return tuple(
                acc[c] + ws * rows[r, pl.ds(c * 16, 16)] for c in range(8)
            )

        acc = lax.fori_loop(
            0, NS, body, tuple(jnp.zeros((16,), _f32) for _ in range(8))
        )
        for c in range(8):
            hp = acc[c] / den_v
            res = jnp.where(hp > 0, hp, jnp.exp(hp) - 1.0)
            outb[b * CHUNK + l, pl.ds(c * 16, 16)] = res


def _sc_stage(h, eluh, neibs_flat, e1f, e2f):
    mesh = plsc.VectorSubcoreMesh(core_axis_name="c", subcore_axis_name="s")
    cp = pltpu.CompilerParams()
    if "needs_layout_passes" in pltpu.CompilerParams.__dataclass_fields__:
        cp = dataclasses.replace(cp, needs_layout_passes=False)

    @functools.partial(
        pl.kernel,
        mesh=mesh,
        compiler_params=cp,
        out_type=jax.ShapeDtypeStruct((N, 2 * DOUT), _f32),
        scratch_types=[
            pltpu.VMEM((NODES_PER_PHASE * NS,), _i32),
            pltpu.VMEM((NODES_PER_TILE,), _f32),
            pltpu.VMEM((NPAD,), _f32),
            pltpu.VMEM((ROWS_PER_CHUNK, DOUT), _f32),
            pltpu.VMEM((ROWS_PER_CHUNK, DOUT), _f32),
            pltpu.VMEM((NS,), _f32),
            pltpu.VMEM((2 * CHUNK, DOUT), _f32),
            pltpu.VMEM_SHARED((N, DOUT), _f32),
            pltpu.SemaphoreType.DMA,
            pltpu.SemaphoreType.DMA,
            pltpu.SemaphoreType.DMA,
        ],
    )
    def sck(h_hbm, eluh_hbm, nb_hbm, e1_hbm, e2_hbm, out_hbm,
            idx_v, e1_v, e2_v, rows0, rows1, watt, outb, h_sh,
            semg0, semg1, semelu):
        sid = lax.axis_index("s")
        cid = lax.axis_index("c")
        wid = sid * 2 + cid
        nbase = wid * NODES_PER_TILE
        elu_lo = sid * 624
        elu_cp = pltpu.make_async_copy(
            eluh_hbm.at[pl.ds(elu_lo, 624)],
            out_hbm.at[pl.ds(elu_lo, 624), pl.ds(0, DOUT)], semelu)
        elu_cp_tail = pltpu.make_async_copy(
            eluh_hbm.at[pl.ds(9984, N - 9984)],
            out_hbm.at[pl.ds(9984, N - 9984), pl.ds(0, DOUT)], semelu)

        @pl.when(cid == 0)
        def _():
            elu_cp.start()

        @pl.when((cid == 0) & (sid == 0))
        def _():
            elu_cp_tail.start()

        pltpu.sync_copy(h_hbm.at[pl.ds(sid * 624, 624)],
                        h_sh.at[pl.ds(sid * 624, 624)])

        @pl.when(sid == 0)
        def _():
            pltpu.sync_copy(h_hbm.at[pl.ds(9984, N - 9984)],
                            h_sh.at[pl.ds(9984, N - 9984)])

        pltpu.sync_copy(e1_hbm.at[pl.ds(nbase, NODES_PER_TILE)], e1_v)
        pltpu.sync_copy(e2_hbm, e2_v)
        plsc.subcore_barrier()

        bufs = (rows0, rows1)
        sems = (semg0, semg1)

        def gather(phase, chunk, buf, sem):
            return pltpu.make_async_copy(
                h_sh.at[idx_v.at[pl.ds(chunk * ROWS_PER_CHUNK,
                                       ROWS_PER_CHUNK)]],
                buf, sem)

        @pl.loop(0, PHASES)
        def _(p):
            pltpu.sync_copy(
                nb_hbm.at[pl.ds((nbase + p * NODES_PER_PHASE) * NS,
                                NODES_PER_PHASE * NS)],
                idx_v)
            gather(p, 0, rows0, semg0).start()

            @pl.loop(0, NCHUNKS, step=2)
            def _(t):
                for b in range(2):
                    chunk = t + b
                    gather(p, chunk, bufs[b], sems[b]).wait()

                    @pl.when(chunk + 1 < NCHUNKS)
                    def _():
                        gather(p, chunk + 1, bufs[1 - b], sems[1 - b]).start()

                    _sc_compute_chunk(bufs[b], idx_v, e1_v, e2_v, watt,
                                      outb, chunk, p, b)
                row0 = nbase + p * NODES_PER_PHASE + t * CHUNK

                @pl.when(row0 + 2 * CHUNK <= N)
                def _():
                    pltpu.sync_copy(
                        outb,
                        out_hbm.at[pl.ds(row0, 2 * CHUNK),
                                   pl.ds(DOUT, DOUT)])

        @pl.when(cid == 0)
        def _():
            elu_cp.wait()

        @pl.when((cid == 0) & (sid == 0))
        def _():
            elu_cp_tail.wait()

    return sck(h, eluh, neibs_flat, e1f, e2f)


def kernel(x, neibs, W, a):
    ap = jnp.zeros((DOUT, 16), _f32)
    ap = ap.at[:, 0].set(a[:DOUT, 0]).at[:, 1].set(a[DOUT:, 0])
    h, e12, eluh = _tc_stage(x, W, ap)

    nb = jnp.pad(neibs.astype(_i32), ((0, NPAD - N), (0, 0)))
    neibs_flat = nb.reshape(NPAD * NS)
    e1f = jnp.pad(e12[:, 0], (0, NPAD - N))
    e2f = jnp.pad(e12[:, 1], (0, NPAD - N))

    return _sc_stage(h, eluh, neibs_flat, e1f, e2f)


# speedup vs baseline: 1.5289x; 1.5289x over previous
import dataclasses
import functools

import jax
import jax.numpy as jnp
from jax import lax
from jax.experimental import pallas as pl
from jax.experimental.pallas import tpu as pltpu
from jax.experimental.pallas import tpu_sc as plsc

N = 10000
NS = 32
DIN = 128
DOUT = 128
ALPHA = 0.2

NTILES = 32
NPAD = 10240
NODES_PER_TILE = NPAD // NTILES
PHASES = 2
NODES_PER_PHASE = NODES_PER_TILE // PHASES
CHUNK = 4
ROWS_PER_CHUNK = CHUNK * NS
NCHUNKS = NODES_PER_PHASE // CHUNK

_f32 = jnp.float32
_i32 = jnp.int32


def _tc_body(x_ref, w_ref, ap_ref, h_ref, e12_ref, eluh_ref):
    h = jnp.dot(x_ref[...], w_ref[...], preferred_element_type=_f32)
    e12 = jnp.dot(h, ap_ref[...], preferred_element_type=_f32)
    h_ref[...] = h
    e12_ref[...] = e12
    eluh_ref[...] = jnp.where(h > 0, h, jnp.exp(h) - 1.0)


def _tc_stage(x, w, ap):
    bm = 2000
    grid = (N // bm,)
    return pl.pallas_call(
        _tc_body,
        grid=grid,
        in_specs=[
            pl.BlockSpec((bm, DIN), lambda i: (i, 0)),
            pl.BlockSpec((DIN, DOUT), lambda i: (0, 0)),
            pl.BlockSpec((DOUT, 16), lambda i: (0, 0)),
        ],
        out_specs=[
            pl.BlockSpec((bm, DOUT), lambda i: (i, 0)),
            pl.BlockSpec((bm, 16), lambda i: (i, 0)),
            pl.BlockSpec((bm, DOUT), lambda i: (i, 0)),
        ],
        out_shape=[
            jax.ShapeDtypeStruct((N, DOUT), _f32),
            jax.ShapeDtypeStruct((N, 16), _f32),
            jax.ShapeDtypeStruct((N, DOUT), _f32),
        ],
    )(x, w, ap)


def _splat_i32(v):
    return jnp.full((16,), v, dtype=_i32)


def _sc_compute_chunk(rows, idx_v, e1_v, e2_v, watt, outb, chunk, phase, b):
    for l in range(CHUNK):
        nip = chunk * CHUNK + l
        nb0 = idx_v[pl.ds(nip * NS, 16)]
        nb1 = idx_v[pl.ds(nip * NS + 16, 16)]
        ev0 = plsc.load_gather(e2_v, [nb0])
        ev1 = plsc.load_gather(e2_v, [nb1])
        e1s = plsc.load_gather(e1_v,
                               [_splat_i32(phase * NODES_PER_PHASE + nip)])
        z0 = e1s + ev0
        z1 = e1s + ev1
        z0 = jnp.maximum(z0, ALPHA * z0)
        z1 = jnp.maximum(z1, ALPHA * z1)
        w0 = jnp.exp(z0)
        w1 = jnp.exp(z1)
        den = jnp.sum(w0 + w1)
        watt[pl.ds(0, 16)] = w0
        watt[pl.ds(16, 16)] = w1
        den_v = jnp.full((16,), den, dtype=_f32)

        def body(s, acc):
            ws = plsc.load_gather(watt, [_splat_i32(s)])
            r = l * NS + s
            return tuple(
                acc[c] + ws * rows[r, pl.ds(c * 16, 16)] for c in range(8)
            )

        acc = lax.fori_loop(
            0, NS, body, tuple(jnp.zeros((16,), _f32) for _ in range(8))
        )
        for c in range(8):
            hp = acc[c] / den_v
            res = jnp.where(hp > 0, hp, jnp.exp(hp) - 1.0)
            outb[b * CHUNK + l, pl.ds(DOUT + c * 16, 16)] = res


def _sc_stage(h, eluh, neibs_flat, e1f, e2f):
    mesh = plsc.VectorSubcoreMesh(core_axis_name="c", subcore_axis_name="s")
    cp = pltpu.CompilerParams()
    if "needs_layout_passes" in pltpu.CompilerParams.__dataclass_fields__:
        cp = dataclasses.replace(cp, needs_layout_passes=False)

    @functools.partial(
        pl.kernel,
        mesh=mesh,
        compiler_params=cp,
        out_type=jax.ShapeDtypeStruct((N, 2 * DOUT), _f32),
        scratch_types=[
            pltpu.VMEM((NODES_PER_PHASE * NS,), _i32),
            pltpu.VMEM((NODES_PER_TILE,), _f32),
            pltpu.VMEM((NPAD,), _f32),
            pltpu.VMEM((ROWS_PER_CHUNK, DOUT), _f32),
            pltpu.VMEM((ROWS_PER_CHUNK, DOUT), _f32),
            pltpu.VMEM((NS,), _f32),
            pltpu.VMEM((2 * CHUNK, 2 * DOUT), _f32),
            pltpu.VMEM_SHARED((N, DOUT), _f32),
            pltpu.SemaphoreType.DMA,
            pltpu.SemaphoreType.DMA,
            pltpu.SemaphoreType.DMA,
        ],
    )
    def sck(h_hbm, eluh_hbm, nb_hbm, e1_hbm, e2_hbm, out_hbm,
            idx_v, e1_v, e2_v, rows0, rows1, watt, outb, h_sh,
            semg0, semg1, semelu):
        sid = lax.axis_index("s")
        cid = lax.axis_index("c")
        wid = sid * 2 + cid
        nbase = wid * NODES_PER_TILE
        pltpu.sync_copy(h_hbm.at[pl.ds(sid * 624, 624)],
                        h_sh.at[pl.ds(sid * 624, 624)])

        @pl.when(sid == 0)
        def _():
            pltpu.sync_copy(h_hbm.at[pl.ds(9984, N - 9984)],
                            h_sh.at[pl.ds(9984, N - 9984)])

        pltpu.sync_copy(e1_hbm.at[pl.ds(nbase, NODES_PER_TILE)], e1_v)
        pltpu.sync_copy(e2_hbm, e2_v)
        plsc.subcore_barrier()

        bufs = (rows0, rows1)
        sems = (semg0, semg1)

        def gather(phase, chunk, buf, sem):
            return pltpu.make_async_copy(
                h_sh.at[idx_v.at[pl.ds(chunk * ROWS_PER_CHUNK,
                                       ROWS_PER_CHUNK)]],
                buf, sem)

        @pl.loop(0, PHASES)
        def _(p):
            pltpu.sync_copy(
                nb_hbm.at[pl.ds((nbase + p * NODES_PER_PHASE) * NS,
                                NODES_PER_PHASE * NS)],
                idx_v)
            gather(p, 0, rows0, semg0).start()

            @pl.loop(0, NCHUNKS, step=2)
            def _(t):
                row0 = nbase + p * NODES_PER_PHASE + t * CHUNK
                valid = row0 + 2 * CHUNK <= N
                elu_cp = pltpu.make_async_copy(
                    eluh_hbm.at[pl.ds(row0, 2 * CHUNK)],
                    outb.at[:, pl.ds(0, DOUT)], semelu)

                @pl.when(valid)
                def _():
                    elu_cp.start()

                for b in range(2):
                    chunk = t + b
                    gather(p, chunk, bufs[b], sems[b]).wait()

                    @pl.when(chunk + 1 < NCHUNKS)
                    def _():
                        gather(p, chunk + 1, bufs[1 - b], sems[1 - b]).start()

                    _sc_compute_chunk(bufs[b], idx_v, e1_v, e2_v, watt,
                                      outb, chunk, p, b)
                @pl.when(valid)
                def _():
                    elu_cp.wait()
                    pltpu.sync_copy(outb,
                                    out_hbm.at[pl.ds(row0, 2 * CHUNK)])

    return sck(h, eluh, neibs_flat, e1f, e2f)


def kernel(x, neibs, W, a):
    ap = jnp.zeros((DOUT, 16), _f32)
    ap = ap.at[:, 0].set(a[:DOUT, 0]).at[:, 1].set(a[DOUT:, 0])
    h, e12, eluh = _tc_stage(x, W, ap)

    nb = jnp.pad(neibs.astype(_i32), ((0, NPAD - N), (0, 0)))
    neibs_flat = nb.reshape(NPAD * NS)
    e1f = jnp.pad(e12[:, 0], (0, NPAD - N))
    e2f = jnp.pad(e12[:, 1], (0, NPAD - N))

    return _sc_stage(h, eluh, neibs_flat, e1f, e2f)
